# Initial kernel scaffold; baseline (speedup 1.0000x reference)
import jax
import jax.numpy as jnp
from jax.experimental import pallas as pl


def kernel(indices, weights, offsets, table):
    raise NotImplementedError("write your pallas kernel here")


# speedup vs baseline: 47.3202x; 47.3202x over previous
import functools

import jax
import jax.numpy as jnp
from jax import lax
from jax.experimental import pallas as pl
from jax.experimental.pallas import tpu as pltpu
from jax.experimental.pallas import tpu_sc as plsc

DIM = 128
NBLK = DIM // 16
CHUNK = 64


def _make_kernel(batch):
    info = plsc.get_sparse_core_info()
    nc, ns = info.num_cores, info.num_subcores
    nw = nc * ns
    bags_per_w = batch // nw
    off_len = bags_per_w + 16

    mesh = plsc.VectorSubcoreMesh(core_axis_name="c", subcore_axis_name="s")

    @functools.partial(
        pl.kernel,
        out_type=jax.ShapeDtypeStruct((batch, DIM), jnp.float32),
        mesh=mesh,
        scratch_types=[
            pltpu.VMEM((off_len,), jnp.int32),
            pltpu.VMEM((CHUNK,), jnp.int32),
            pltpu.VMEM((CHUNK,), jnp.float32),
            pltpu.VMEM((CHUNK, DIM), jnp.float32),
            pltpu.VMEM((bags_per_w, DIM), jnp.float32),
            pltpu.SemaphoreType.DMA,
        ],
    )
    def kern(idx_hbm, w_hbm, off_hbm, tab_hbm, out_hbm,
             off_v, idx_v, w_v, rows_v, out_v, sem):
        wid = lax.axis_index("s") * nc + lax.axis_index("c")
        bag0 = wid * bags_per_w
        pltpu.sync_copy(off_hbm.at[pl.ds(bag0, off_len)], off_v)

        def bag_body(b, _):
            off_pair = off_v[pl.ds(b, 16)]
            start = off_pair[0]
            end = off_pair[1]
            abase = pl.multiple_of(start - lax.rem(start, 8), 8)
            nchunks = (end - abase + (CHUNK - 1)) // CHUNK

            def chunk_body(k, acc):
                pos = pl.multiple_of(abase + k * CHUNK, 8)
                pltpu.sync_copy(idx_hbm.at[pl.ds(pos, CHUNK)], idx_v)
                pltpu.sync_copy(w_hbm.at[pl.ds(pos, CHUNK)], w_v)
                pltpu.async_copy(tab_hbm.at[idx_v], rows_v, sem).wait()
                acc = list(acc)
                for sg in range(CHUNK // 16):
                    wvec = w_v[pl.ds(sg * 16, 16)]
                    for e16 in range(16):
                        e = sg * 16 + e16
                        p = pos + e
                        w_s = jnp.where((p >= start) & (p < end), wvec[e16], 0.0)
                        w_b = jnp.zeros((16,), jnp.float32) + w_s
                        for d in range(NBLK):
                            acc[d] = acc[d] + w_b * rows_v[e, pl.ds(d * 16, 16)]
                return tuple(acc)

            acc0 = tuple(jnp.zeros((16,), jnp.float32) for _ in range(NBLK))
            acc = lax.fori_loop(0, nchunks, chunk_body, acc0)
            for d in range(NBLK):
                out_v[b, pl.ds(d * 16, 16)] = acc[d]
            return 0

        lax.fori_loop(0, bags_per_w, bag_body, 0)
        pltpu.sync_copy(out_v, out_hbm.at[pl.ds(bag0, bags_per_w), :])

    return kern


@jax.jit
def kernel(indices, weights, offsets, table):
    batch = offsets.shape[0] - 1
    total = indices.shape[0]
    pad = 2 * CHUNK
    idx_p = jnp.pad(indices.astype(jnp.int32), (0, pad))
    w_p = jnp.pad(weights, (0, pad))
    off_p = jnp.pad(offsets, (0, 256), constant_values=total)
    kern = _make_kernel(batch)
    return kern(idx_p, w_p, off_p, table)


# speedup vs baseline: 242.5880x; 5.1265x over previous
import functools

import jax
import jax.numpy as jnp
from jax import lax
from jax.experimental import pallas as pl
from jax.experimental.pallas import tpu as pltpu
from jax.experimental.pallas import tpu_sc as plsc

DIM = 128
NBLK = DIM // 16
CHUNK = 128
GPC = CHUNK // 16
NBUF = 4
NBUF2 = 8


def _make_kernel(batch):
    info = plsc.get_sparse_core_info()
    nc, ns = info.num_cores, info.num_subcores
    nw = nc * ns
    bags_per_w = batch // nw
    off_len = bags_per_w + 16

    mesh = plsc.VectorSubcoreMesh(core_axis_name="c", subcore_axis_name="s")

    @functools.partial(
        pl.kernel,
        out_type=jax.ShapeDtypeStruct((batch, DIM), jnp.float32),
        mesh=mesh,
        scratch_types=[
            pltpu.VMEM((off_len,), jnp.int32),
            pltpu.VMEM((NBUF2 * CHUNK,), jnp.int32),
            pltpu.VMEM((NBUF2 * CHUNK,), jnp.float32),
            pltpu.VMEM((NBUF * CHUNK, DIM), jnp.float32),
            pltpu.VMEM((bags_per_w, DIM), jnp.float32),
            pltpu.SemaphoreType.DMA((NBUF2,)),
            pltpu.SemaphoreType.DMA((NBUF,)),
        ],
    )
    def kern(idx_hbm, w_hbm, off_hbm, tab_hbm, out_hbm,
             off_v, idx_v, w_v, rows_v, out_v, iwsem, gsem):
        wid = lax.axis_index("s") * nc + lax.axis_index("c")
        bag0 = wid * bags_per_w
        pltpu.sync_copy(off_hbm.at[pl.ds(bag0, off_len)], off_v)

        head = off_v[pl.ds(0, 16)]
        start_w = head[0]
        wbase = pl.multiple_of(start_w - lax.rem(start_w, 8), 8)
        tail = off_v[pl.ds(bags_per_w, 16)]
        end_w = tail[0]
        nch = (end_w - wbase + (CHUNK - 1)) // CHUNK

        def issue_iw(j):
            slot = lax.rem(j, NBUF2)
            pos = pl.multiple_of(wbase + j * CHUNK, 8)
            dst = pl.ds(slot * CHUNK, CHUNK)
            pltpu.async_copy(idx_hbm.at[pl.ds(pos, CHUNK)], idx_v.at[dst],
                             iwsem.at[slot])
            pltpu.async_copy(w_hbm.at[pl.ds(pos, CHUNK)], w_v.at[dst],
                             iwsem.at[slot])

        def wait_iw(j):
            slot = lax.rem(j, NBUF2)
            dst = pl.ds(slot * CHUNK, CHUNK)
            pltpu.make_async_copy(idx_hbm.at[pl.ds(0, CHUNK)], idx_v.at[dst],
                                  iwsem.at[slot]).wait()
            pltpu.make_async_copy(w_hbm.at[pl.ds(0, CHUNK)], w_v.at[dst],
                                  iwsem.at[slot]).wait()

        def issue_gather(j):
            slot = lax.rem(j, NBUF)
            slot2 = lax.rem(j, NBUF2)
            pltpu.async_copy(
                tab_hbm.at[idx_v.at[pl.ds(slot2 * CHUNK, CHUNK)]],
                rows_v.at[pl.ds(slot * CHUNK, CHUNK), :],
                gsem.at[slot])

        def wait_gather(j):
            slot = lax.rem(j, NBUF)
            pltpu.make_async_copy(tab_hbm.at[pl.ds(0, CHUNK), :],
                                  rows_v.at[pl.ds(slot * CHUNK, CHUNK), :],
                                  gsem.at[slot]).wait()

        def need(c, iw, gi, gw):
            iw_hi = jnp.minimum(c + 2 * NBUF, nch)
            lax.fori_loop(iw, iw_hi, lambda j, _: (issue_iw(j), 0)[1], 0)
            iw = jnp.maximum(iw, iw_hi)

            gi_hi = jnp.minimum(c + NBUF, nch)
            lax.fori_loop(
                gi, gi_hi,
                lambda j, _: (wait_iw(j), issue_gather(j), 0)[2], 0)
            gi = jnp.maximum(gi, gi_hi)

            lax.fori_loop(gw, c + 1, lambda j, _: (wait_gather(j), 0)[1], 0)
            gw = jnp.maximum(gw, c + 1)
            return iw, gi, gw

        lanes = lax.iota(jnp.int32, 16)

        def bag_body(b, st):
            iw, gi, gw = st
            off_pair = off_v[pl.ds(b, 16)]
            s = off_pair[0]
            e = off_pair[1]
            gs = (s - wbase) // 16
            ge1 = (e - wbase + 15) // 16

            def g_body(g, carry):
                acc = list(carry[:NBLK])
                iw, gi, gw = carry[NBLK:]
                c = g // GPC
                iw, gi, gw = need(c, iw, gi, gw)
                go = g - c * GPC
                slot_off = lax.rem(c, NBUF2) * CHUNK + go * 16
                wvec = w_v[pl.ds(pl.multiple_of(slot_off, 16), 16)]
                p = wbase + g * 16 + lanes
                wm = jnp.where((p >= s) & (p < e), wvec, 0.0)
                rbase = lax.rem(c, NBUF) * CHUNK + go * 16
                for e16 in range(16):
                    w_b = jnp.zeros((16,), jnp.float32) + wm[e16]
                    row = rbase + e16
                    for d in range(NBLK):
                        acc[d] = acc[d] + w_b * rows_v[row, pl.ds(d * 16, 16)]
                return tuple(acc) + (iw, gi, gw)

            acc0 = tuple(jnp.zeros((16,), jnp.float32) for _ in range(NBLK))
            res = lax.fori_loop(gs, ge1, g_body, acc0 + (iw, gi, gw))
            for d in range(NBLK):
                out_v[b, pl.ds(d * 16, 16)] = res[d]
            return res[NBLK:]

        z = jnp.int32(0)
        lax.fori_loop(0, bags_per_w, bag_body, (z, z, z))
        pltpu.sync_copy(out_v, out_hbm.at[pl.ds(bag0, bags_per_w), :])

    return kern


@jax.jit
def kernel(indices, weights, offsets, table):
    batch = offsets.shape[0] - 1
    total = indices.shape[0]
    pad = 2 * CHUNK
    idx_p = jnp.pad(indices.astype(jnp.int32), (0, pad))
    w_p = jnp.pad(weights, (0, pad))
    off_p = jnp.pad(offsets, (0, 256), constant_values=total)
    kern = _make_kernel(batch)
    return kern(idx_p, w_p, off_p, table)


# speedup vs baseline: 257.2198x; 1.0603x over previous
import functools

import jax
import jax.numpy as jnp
from jax import lax
from jax.experimental import pallas as pl
from jax.experimental.pallas import tpu as pltpu
from jax.experimental.pallas import tpu_sc as plsc

DIM = 128
NBLK = DIM // 16
CHUNK = 128
GPC = CHUNK // 16
NBUF = 4
NBUF2 = 8


def _make_kernel(batch):
    info = plsc.get_sparse_core_info()
    nc, ns = info.num_cores, info.num_subcores
    nw = nc * ns
    bags_per_w = batch // nw
    off_len = bags_per_w + 16

    mesh = plsc.VectorSubcoreMesh(core_axis_name="c", subcore_axis_name="s")

    @functools.partial(
        pl.kernel,
        out_type=jax.ShapeDtypeStruct((batch, DIM), jnp.float32),
        mesh=mesh,
        scratch_types=[
            pltpu.VMEM((off_len,), jnp.int32),
            pltpu.VMEM((NBUF2 * CHUNK,), jnp.int32),
            pltpu.VMEM((NBUF2 * CHUNK,), jnp.float32),
            pltpu.VMEM((NBUF * CHUNK, DIM), jnp.float32),
            pltpu.VMEM((bags_per_w, DIM), jnp.float32),
            pltpu.SemaphoreType.DMA((NBUF2,)),
            pltpu.SemaphoreType.DMA((NBUF,)),
        ],
    )
    def kern(idx_hbm, w_hbm, off_hbm, tab_hbm, out_hbm,
             off_v, idx_v, w_v, rows_v, out_v, iwsem, gsem):
        wid = lax.axis_index("s") * nc + lax.axis_index("c")
        bag0 = wid * bags_per_w
        pltpu.sync_copy(off_hbm.at[pl.ds(bag0, off_len)], off_v)

        head = off_v[pl.ds(0, 16)]
        start_w = head[0]
        wbase = pl.multiple_of(start_w - lax.rem(start_w, 8), 8)
        tail = off_v[pl.ds(bags_per_w, 16)]
        end_w = tail[0]
        nch = (end_w - wbase + (CHUNK - 1)) // CHUNK

        def issue_iw(j):
            slot = lax.rem(j, NBUF2)
            pos = pl.multiple_of(wbase + j * CHUNK, 8)
            dst = pl.ds(slot * CHUNK, CHUNK)
            pltpu.async_copy(idx_hbm.at[pl.ds(pos, CHUNK)], idx_v.at[dst],
                             iwsem.at[slot])
            pltpu.async_copy(w_hbm.at[pl.ds(pos, CHUNK)], w_v.at[dst],
                             iwsem.at[slot])

        def wait_iw(j):
            slot = lax.rem(j, NBUF2)
            dst = pl.ds(slot * CHUNK, CHUNK)
            pltpu.make_async_copy(idx_hbm.at[pl.ds(0, CHUNK)], idx_v.at[dst],
                                  iwsem.at[slot]).wait()
            pltpu.make_async_copy(w_hbm.at[pl.ds(0, CHUNK)], w_v.at[dst],
                                  iwsem.at[slot]).wait()

        def issue_gather(j):
            slot = lax.rem(j, NBUF)
            slot2 = lax.rem(j, NBUF2)
            pltpu.async_copy(
                tab_hbm.at[idx_v.at[pl.ds(slot2 * CHUNK, CHUNK)]],
                rows_v.at[pl.ds(slot * CHUNK, CHUNK), :],
                gsem.at[slot])

        def wait_gather(j):
            slot = lax.rem(j, NBUF)
            pltpu.make_async_copy(tab_hbm.at[pl.ds(0, CHUNK), :],
                                  rows_v.at[pl.ds(slot * CHUNK, CHUNK), :],
                                  gsem.at[slot]).wait()

        def need(c, iw, gi, gw):
            iw_hi = jnp.minimum(c + 2 * NBUF, nch)
            lax.fori_loop(iw, iw_hi, lambda j, _: (issue_iw(j), 0)[1], 0)
            iw = jnp.maximum(iw, iw_hi)

            gi_hi = jnp.minimum(c + NBUF, nch)
            lax.fori_loop(
                gi, gi_hi,
                lambda j, _: (wait_iw(j), issue_gather(j), 0)[2], 0)
            gi = jnp.maximum(gi, gi_hi)

            lax.fori_loop(gw, c + 1, lambda j, _: (wait_gather(j), 0)[1], 0)
            gw = jnp.maximum(gw, c + 1)
            return iw, gi, gw

        lanes = lax.iota(jnp.int32, 16)

        def bag_body(b, st):
            iw, gi, gw = st
            off_pair = off_v[pl.ds(b, 16)]
            s = off_pair[0]
            e = off_pair[1]
            gs = (s - wbase) // 16
            ge1 = (e - wbase + 15) // 16

            def g_body(g, carry):
                acc = list(carry[:NBLK])
                iw, gi, gw = carry[NBLK:]
                c = g // GPC
                iw, gi, gw = lax.cond(
                    gw <= c,
                    lambda: need(c, iw, gi, gw),
                    lambda: (iw, gi, gw))
                go = g - c * GPC
                slot_off = lax.rem(c, NBUF2) * CHUNK + go * 16
                wvec = w_v[pl.ds(pl.multiple_of(slot_off, 16), 16)]
                p = wbase + g * 16 + lanes
                wm = jnp.where((p >= s) & (p < e), wvec, 0.0)
                rbase = lax.rem(c, NBUF) * CHUNK + go * 16
                for e16 in range(16):
                    w_b = jnp.zeros((16,), jnp.float32) + wm[e16]
                    row = rbase + e16
                    for d in range(NBLK):
                        acc[d] = acc[d] + w_b * rows_v[row, pl.ds(d * 16, 16)]
                return tuple(acc) + (iw, gi, gw)

            acc0 = tuple(jnp.zeros((16,), jnp.float32) for _ in range(NBLK))
            res = lax.fori_loop(gs, ge1, g_body, acc0 + (iw, gi, gw))
            for d in range(NBLK):
                out_v[b, pl.ds(d * 16, 16)] = res[d]
            return res[NBLK:]

        z = jnp.int32(0)
        lax.fori_loop(0, bags_per_w, bag_body, (z, z, z))
        pltpu.sync_copy(out_v, out_hbm.at[pl.ds(bag0, bags_per_w), :])

    return kern


@jax.jit
def kernel(indices, weights, offsets, table):
    batch = offsets.shape[0] - 1
    total = indices.shape[0]
    pad = 2 * CHUNK
    idx_p = jnp.pad(indices.astype(jnp.int32), (0, pad))
    w_p = jnp.pad(weights, (0, pad))
    off_p = jnp.pad(offsets, (0, 256), constant_values=total)
    kern = _make_kernel(batch)
    return kern(idx_p, w_p, off_p, table)


# speedup vs baseline: 264.8121x; 1.0295x over previous
import functools

import jax
import jax.numpy as jnp
from jax import lax
from jax.experimental import pallas as pl
from jax.experimental.pallas import tpu as pltpu
from jax.experimental.pallas import tpu_sc as plsc

DIM = 128
NBLK = DIM // 16
CHUNK = 128
GPC = CHUNK // 16
NBUF = 4
NBUF2 = 8


def _make_kernel(batch):
    info = plsc.get_sparse_core_info()
    nc, ns = info.num_cores, info.num_subcores
    nw = nc * ns
    bags_per_w = batch // nw
    off_len = bags_per_w + 16

    mesh = plsc.VectorSubcoreMesh(core_axis_name="c", subcore_axis_name="s")

    @functools.partial(
        pl.kernel,
        out_type=jax.ShapeDtypeStruct((batch, DIM), jnp.float32),
        mesh=mesh,
        scratch_types=[
            pltpu.VMEM((off_len,), jnp.int32),
            pltpu.VMEM((NBUF2 * CHUNK,), jnp.int32),
            pltpu.VMEM((NBUF2 * CHUNK,), jnp.float32),
            pltpu.VMEM((NBUF * CHUNK, DIM), jnp.float32),
            pltpu.VMEM((bags_per_w, DIM), jnp.float32),
            pltpu.VMEM((DIM,), jnp.float32),
            pltpu.SemaphoreType.DMA((NBUF2,)),
            pltpu.SemaphoreType.DMA((NBUF,)),
        ],
    )
    def kern(idx_hbm, w_hbm, off_hbm, tab_hbm, out_hbm,
             off_v, idx_v, w_v, rows_v, out_v, car_v, iwsem, gsem):
        wid = lax.axis_index("s") * nc + lax.axis_index("c")
        bag0 = wid * bags_per_w
        pltpu.sync_copy(off_hbm.at[pl.ds(bag0, off_len)], off_v)

        head = off_v[pl.ds(0, 16)]
        start_w = head[0]
        e0 = head[1]
        wbase = pl.multiple_of(start_w - lax.rem(start_w, 8), 8)
        tail_v = off_v[pl.ds(bags_per_w, 16)]
        end_w = tail_v[0]
        nch = (end_w - wbase + (CHUNK - 1)) // CHUNK

        def issue_iw(j):
            slot = lax.rem(j, NBUF2)
            pos = pl.multiple_of(wbase + j * CHUNK, 8)
            dst = pl.ds(slot * CHUNK, CHUNK)
            pltpu.async_copy(idx_hbm.at[pl.ds(pos, CHUNK)], idx_v.at[dst],
                             iwsem.at[slot])
            pltpu.async_copy(w_hbm.at[pl.ds(pos, CHUNK)], w_v.at[dst],
                             iwsem.at[slot])

        def wait_iw(j):
            slot = lax.rem(j, NBUF2)
            dst = pl.ds(slot * CHUNK, CHUNK)
            pltpu.make_async_copy(idx_hbm.at[pl.ds(0, CHUNK)], idx_v.at[dst],
                                  iwsem.at[slot]).wait()
            pltpu.make_async_copy(w_hbm.at[pl.ds(0, CHUNK)], w_v.at[dst],
                                  iwsem.at[slot]).wait()

        def issue_gather(j):
            slot = lax.rem(j, NBUF)
            slot2 = lax.rem(j, NBUF2)
            pltpu.async_copy(
                tab_hbm.at[idx_v.at[pl.ds(slot2 * CHUNK, CHUNK)]],
                rows_v.at[pl.ds(slot * CHUNK, CHUNK), :],
                gsem.at[slot])

        def wait_gather(j):
            slot = lax.rem(j, NBUF)
            pltpu.make_async_copy(tab_hbm.at[pl.ds(0, CHUNK), :],
                                  rows_v.at[pl.ds(slot * CHUNK, CHUNK), :],
                                  gsem.at[slot]).wait()

        def need(c, iw, gi, gw):
            iw_hi = jnp.minimum(c + 2 * NBUF, nch)
            lax.fori_loop(iw, iw_hi, lambda j, _: (issue_iw(j), 0)[1], 0)
            iw = jnp.maximum(iw, iw_hi)

            gi_hi = jnp.minimum(c + NBUF, nch)
            lax.fori_loop(
                gi, gi_hi,
                lambda j, _: (wait_iw(j), issue_gather(j), 0)[2], 0)
            gi = jnp.maximum(gi, gi_hi)

            lax.fori_loop(gw, c + 1, lambda j, _: (wait_gather(j), 0)[1], 0)
            gw = jnp.maximum(gw, c + 1)
            return iw, gi, gw

        def need_if(c, iw, gi, gw):
            return lax.cond(gw <= c, lambda: need(c, iw, gi, gw),
                            lambda: (iw, gi, gw))

        lanes = lax.iota(jnp.int32, 16)

        def group_addrs(g):
            c = g // GPC
            go = g - c * GPC
            woff = lax.rem(c, NBUF2) * CHUNK + go * 16
            rbase = lax.rem(c, NBUF) * CHUNK + go * 16
            return c, woff, rbase

        def dual_group(wm_a, wm_b, rbase, acc, car):
            acc = list(acc)
            car = list(car)
            for e16 in range(16):
                wa = jnp.zeros((16,), jnp.float32) + wm_a[e16]
                wb = jnp.zeros((16,), jnp.float32) + wm_b[e16]
                row = rbase + e16
                for d in range(NBLK):
                    blk = rows_v[row, pl.ds(d * 16, 16)]
                    acc[d] = acc[d] + wa * blk
                    car[d] = car[d] + wb * blk
            return tuple(acc), tuple(car)

        zacc = tuple(jnp.zeros((16,), jnp.float32) for _ in range(NBLK))

        def store_car(car):
            for d in range(NBLK):
                car_v[pl.ds(d * 16, 16)] = car[d]

        def prologue():
            gp = (start_w - wbase) // 16
            glo0 = (start_w - wbase + 15) // 16
            iw, gi, gw = need(gp // GPC, jnp.int32(0), jnp.int32(0),
                              jnp.int32(0))
            c, woff, rbase = group_addrs(gp)
            wvec = w_v[pl.ds(pl.multiple_of(woff, 16), 16)]
            p = wbase + gp * 16 + lanes
            lim = jnp.minimum(e0, wbase + 16 * glo0)
            wm = jnp.where((p >= start_w) & (p < lim), wvec, 0.0)
            _, car = dual_group(wm, wm, rbase, zacc, zacc)
            store_car(car)
            return iw, gi, gw

        def no_prologue():
            z = jnp.int32(0)
            store_car(zacc)
            return z, z, z

        pro = lax.cond(nch > 0, prologue, no_prologue)

        def bag_body(b, st):
            iw, gi, gw = st
            car = tuple(car_v[pl.ds(d * 16, 16)] for d in range(NBLK))
            offv = off_v[pl.ds(b, 16)]
            s = offv[0]
            e = offv[1]
            e2 = offv[2]
            g_lo = (s - wbase + 15) // 16
            ge1 = (e - wbase + 15) // 16
            gt = ge1 - 1

            def g_body(g, carry):
                acc = list(carry[:NBLK])
                iw, gi, gw = carry[NBLK:]
                c, woff, rbase = group_addrs(g)
                iw, gi, gw = need_if(c, iw, gi, gw)
                wvec = w_v[pl.ds(pl.multiple_of(woff, 16), 16)]
                for e16 in range(16):
                    w_b = jnp.zeros((16,), jnp.float32) + wvec[e16]
                    row = rbase + e16
                    for d in range(NBLK):
                        acc[d] = acc[d] + w_b * rows_v[row, pl.ds(d * 16, 16)]
                return tuple(acc) + (iw, gi, gw)

            res = lax.fori_loop(g_lo, jnp.maximum(gt, g_lo), g_body,
                                car + (iw, gi, gw))
            acc = res[:NBLK]
            iw, gi, gw = res[NBLK:]
            iw, gi, gw = need_if(gt // GPC, iw, gi, gw)

            def tail():
                c, woff, rbase = group_addrs(gt)
                tiw, tgi, tgw = iw, gi, gw
                wvec = w_v[pl.ds(pl.multiple_of(woff, 16), 16)]
                p = wbase + gt * 16 + lanes
                e_eff = jnp.where(gt >= g_lo, e, jnp.int32(-1))
                wm_s = jnp.where(p < e_eff, wvec, 0.0)
                wm_n = jnp.where((p >= e) & (p < e2), wvec, 0.0)
                nacc, ncar = dual_group(wm_s, wm_n, rbase, acc, zacc)
                for d in range(NBLK):
                    out_v[b, pl.ds(d * 16, 16)] = nacc[d]
                store_car(ncar)
                return tiw, tgi, tgw

            def no_tail():
                for d in range(NBLK):
                    out_v[b, pl.ds(d * 16, 16)] = acc[d]
                store_car(zacc)
                return iw, gi, gw

            return lax.cond(gt >= 0, tail, no_tail)

        lax.fori_loop(0, bags_per_w, bag_body, pro)
        pltpu.sync_copy(out_v, out_hbm.at[pl.ds(bag0, bags_per_w), :])

    return kern


@jax.jit
def kernel(indices, weights, offsets, table):
    batch = offsets.shape[0] - 1
    total = indices.shape[0]
    pad = 2 * CHUNK
    idx_p = jnp.pad(indices.astype(jnp.int32), (0, pad))
    w_p = jnp.pad(weights, (0, pad))
    off_p = jnp.pad(offsets, (0, 256), constant_values=total)
    kern = _make_kernel(batch)
    return kern(idx_p, w_p, off_p, table)
